# baseline (device time: 99139 ns/iter reference)
import jax
import jax.numpy as jnp
from jax import lax
from jax.experimental import pallas as pl
from jax.experimental.pallas import tpu as pltpu

T = 1024
D = 1024
F = 2048
E_LOCAL = 2


def kernel(x, assign, W1, W2):
    xb = x.astype(jnp.bfloat16)
    w1b = W1.astype(jnp.bfloat16)
    w2b = W2.astype(jnp.bfloat16)
    a2d = assign.reshape(T, 1)

    def body(x_ref, a_ref, w1_ref, w2_ref, out_ref,
             xrecv, arecv, pbuf, precv, send_sems, recv_sems):
        my_x = lax.axis_index("x")
        my_y = lax.axis_index("y")
        my_z = lax.axis_index("z")
        peer = (my_x, my_y, 1 - my_z)

        barrier = pltpu.get_barrier_semaphore()
        pl.semaphore_signal(barrier, inc=1, device_id=peer,
                            device_id_type=pl.DeviceIdType.MESH)
        pl.semaphore_wait(barrier, 1)

        rx = pltpu.make_async_remote_copy(
            src_ref=x_ref, dst_ref=xrecv,
            send_sem=send_sems.at[0], recv_sem=recv_sems.at[0],
            device_id=peer, device_id_type=pl.DeviceIdType.MESH)
        ra = pltpu.make_async_remote_copy(
            src_ref=a_ref, dst_ref=arecv,
            send_sem=send_sems.at[1], recv_sem=recv_sems.at[1],
            device_id=peer, device_id_type=pl.DeviceIdType.MESH)
        rx.start()
        ra.start()

        def local_experts(xt, at):
            acc = jnp.zeros((T, D), jnp.float32)
            for el in range(E_LOCAL):
                ge = my_z * E_LOCAL + el
                xm = jnp.where(at == ge, xt, 0).astype(jnp.bfloat16)
                h = jnp.dot(xm, w1_ref[el],
                            preferred_element_type=jnp.float32)
                h = jnp.maximum(h, 0.0).astype(jnp.bfloat16)
                acc = acc + jnp.dot(h, w2_ref[el],
                                    preferred_element_type=jnp.float32)
            return acc

        own = local_experts(x_ref[:, :], a_ref[:, :])

        rx.wait()
        ra.wait()

        pbuf[:, :] = local_experts(xrecv[:, :], arecv[:, :]).astype(jnp.bfloat16)

        rp = pltpu.make_async_remote_copy(
            src_ref=pbuf, dst_ref=precv,
            send_sem=send_sems.at[2], recv_sem=recv_sems.at[2],
            device_id=peer, device_id_type=pl.DeviceIdType.MESH)
        rp.start()
        rp.wait()

        out_ref[:, :] = own + precv[:, :].astype(jnp.float32)

    return pl.pallas_call(
        body,
        out_shape=jax.ShapeDtypeStruct((T, D), jnp.float32),
        in_specs=[pl.BlockSpec(memory_space=pltpu.VMEM)] * 4,
        out_specs=pl.BlockSpec(memory_space=pltpu.VMEM),
        scratch_shapes=[
            pltpu.VMEM((T, D), jnp.bfloat16),
            pltpu.VMEM((T, 1), jnp.int32),
            pltpu.VMEM((T, D), jnp.bfloat16),
            pltpu.VMEM((T, D), jnp.bfloat16),
            pltpu.SemaphoreType.DMA((3,)),
            pltpu.SemaphoreType.DMA((3,)),
        ],
        compiler_params=pltpu.CompilerParams(collective_id=0),
    )(xb, a2d, w1b, w2b)


# device time: 85617 ns/iter; 1.1579x vs baseline; 1.1579x over previous
import jax
import jax.numpy as jnp
from jax import lax
from jax.experimental import pallas as pl
from jax.experimental.pallas import tpu as pltpu

T = 1024
D = 1024
F = 2048
E_LOCAL = 2
NC = 4
CH = T // NC


def kernel(x, assign, W1, W2):
    xb = x.astype(jnp.bfloat16)
    w1b = W1.astype(jnp.bfloat16)
    w2b = W2.astype(jnp.bfloat16)
    a2d = assign.reshape(T, 1)

    def body(x_ref, a_ref, w1_ref, w2_ref, out_ref,
             xrecv, arecv, pbuf, precv, send_sems, recv_sems):
        my_x = lax.axis_index("x")
        my_y = lax.axis_index("y")
        my_z = lax.axis_index("z")
        peer = (my_x, my_y, 1 - my_z)

        barrier = pltpu.get_barrier_semaphore()
        pl.semaphore_signal(barrier, inc=1, device_id=peer,
                            device_id_type=pl.DeviceIdType.MESH)
        pl.semaphore_wait(barrier, 1)

        rx = pltpu.make_async_remote_copy(
            src_ref=x_ref, dst_ref=xrecv,
            send_sem=send_sems.at[0], recv_sem=recv_sems.at[0],
            device_id=peer, device_id_type=pl.DeviceIdType.MESH)
        ra = pltpu.make_async_remote_copy(
            src_ref=a_ref, dst_ref=arecv,
            send_sem=send_sems.at[1], recv_sem=recv_sems.at[1],
            device_id=peer, device_id_type=pl.DeviceIdType.MESH)
        rx.start()
        ra.start()

        def local_experts(xt, at, rows):
            acc = jnp.zeros((rows, D), jnp.float32)
            for el in range(E_LOCAL):
                ge = my_z * E_LOCAL + el
                xm = jnp.where(at == ge, xt, 0).astype(jnp.bfloat16)
                h = jnp.dot(xm, w1_ref[el],
                            preferred_element_type=jnp.float32)
                h = jnp.maximum(h, 0.0).astype(jnp.bfloat16)
                acc = acc + jnp.dot(h, w2_ref[el],
                                    preferred_element_type=jnp.float32)
            return acc

        out_ref[:, :] = local_experts(x_ref[:, :], a_ref[:, :], T)

        rx.wait()
        ra.wait()

        rps = []
        for c in range(NC):
            sl = slice(c * CH, (c + 1) * CH)
            pbuf[c] = local_experts(xrecv[sl, :], arecv[sl, :], CH).astype(
                jnp.bfloat16)
            rp = pltpu.make_async_remote_copy(
                src_ref=pbuf.at[c], dst_ref=precv.at[c],
                send_sem=send_sems.at[2 + c], recv_sem=recv_sems.at[2 + c],
                device_id=peer, device_id_type=pl.DeviceIdType.MESH)
            rp.start()
            rps.append(rp)

        for c in range(NC):
            rps[c].wait_recv()
            sl = slice(c * CH, (c + 1) * CH)
            out_ref[sl, :] = out_ref[sl, :] + precv[c].astype(jnp.float32)
        for c in range(NC):
            rps[c].wait_send()

    return pl.pallas_call(
        body,
        out_shape=jax.ShapeDtypeStruct((T, D), jnp.float32),
        in_specs=[pl.BlockSpec(memory_space=pltpu.VMEM)] * 4,
        out_specs=pl.BlockSpec(memory_space=pltpu.VMEM),
        scratch_shapes=[
            pltpu.VMEM((T, D), jnp.bfloat16),
            pltpu.VMEM((T, 1), jnp.int32),
            pltpu.VMEM((NC, CH, D), jnp.bfloat16),
            pltpu.VMEM((NC, CH, D), jnp.bfloat16),
            pltpu.SemaphoreType.DMA((2 + NC,)),
            pltpu.SemaphoreType.DMA((2 + NC,)),
        ],
        compiler_params=pltpu.CompilerParams(collective_id=0),
    )(xb, a2d, w1b, w2b)
